# baseline (speedup 1.0000x reference)
import functools

import jax
import jax.numpy as jnp
from jax import lax
from jax.experimental import pallas as pl
from jax.experimental.pallas import tpu as pltpu
from jax.experimental.pallas import tpu_sc as plsc

N = 10000
D = 128
E = 320000

NC = 2
NS = 16
NW = NC * NS
B = 128
CH = 80
CAP = NW * CH * B
NPR = N + 8
NPD = NS * 8 * 128
DPT = NPD // NS
RPT = (N // NS) // 8 * 8
TAIL = N - NS * RPT

_mesh = plsc.VectorSubcoreMesh(core_axis_name="c", subcore_axis_name="s")


@functools.partial(
    pl.kernel,
    out_type=jax.ShapeDtypeStruct((NC, NPD // 128, 128), jnp.float32),
    mesh=_mesh,
    scratch_types=[
        pltpu.VMEM((CH, B), jnp.int32),
        pltpu.VMEM((DPT,), jnp.float32),
        pltpu.VMEM((B,), jnp.float32),
        pltpu.VMEM((DPT // 128, 128), jnp.float32),
        pltpu.VMEM_SHARED((NPD,), jnp.float32),
    ],
)
def _sc_degree(dstp, out, dst_v, ones_i, ones_b, stage_v, deg_sh):
    c = lax.axis_index("c")
    s = lax.axis_index("s")
    wid = s * NC + c
    for i in range(DPT // 16):
        ones_i[pl.ds(i * 16, 16)] = jnp.full((16,), 1.0, jnp.float32)
    for i in range(B // 16):
        ones_b[pl.ds(i * 16, 16)] = jnp.full((16,), 1.0, jnp.float32)
    pltpu.sync_copy(dstp.at[wid], dst_v)
    pltpu.sync_copy(ones_i, deg_sh.at[pl.ds(s * DPT, DPT)])
    plsc.subcore_barrier()

    def body(j, carry):
        pltpu.sync_copy(ones_b, deg_sh.at[dst_v.at[j]], add=True)
        return carry

    lax.fori_loop(0, CH, body, 0)
    plsc.subcore_barrier()
    for t in range(DPT // 128):
        pltpu.sync_copy(deg_sh.at[pl.ds(s * DPT + t * 128, 128)], stage_v.at[t])
    pltpu.sync_copy(stage_v, out.at[c].at[pl.ds(s * (DPT // 128), DPT // 128)])


@functools.partial(
    pl.kernel,
    out_type=jax.ShapeDtypeStruct((NC, N, D), jnp.float32),
    mesh=_mesh,
    scratch_types=[
        pltpu.VMEM((CH, B), jnp.int32),
        [pltpu.VMEM((B,), jnp.int32) for _ in range(2)],
        [pltpu.VMEM((B,), jnp.int32) for _ in range(2)],
        [pltpu.VMEM((B, D), jnp.float32) for _ in range(2)],
        pltpu.VMEM_SHARED((NPR, D), jnp.float32),
        [pltpu.SemaphoreType.DMA for _ in range(2)],
    ],
)
def _sc_aggregate(t_hbm, pkp, out, pk_v, srcu, dstu, rows, acc_sh, sg):
    c = lax.axis_index("c")
    s = lax.axis_index("s")
    wid = s * NC + c
    pltpu.sync_copy(t_hbm.at[pl.ds(s * RPT, RPT)], acc_sh.at[pl.ds(s * RPT, RPT)])

    @pl.when(s == NS - 1)
    def _init_tail():
        pltpu.sync_copy(t_hbm.at[pl.ds(NS * RPT, TAIL)],
                        acc_sh.at[pl.ds(NS * RPT, TAIL)])

    def _wait(buf, sem):
        pltpu.make_async_copy(t_hbm.at[pl.ds(0, B)], buf, sem).wait()

    def _unpack(j, k):
        for i in range(B // 16):
            w = pk_v[j, pl.ds(i * 16, 16)]
            srcu[k][pl.ds(i * 16, 16)] = lax.shift_right_logical(w, 14)
            dstu[k][pl.ds(i * 16, 16)] = lax.bitwise_and(
                w, jnp.full((16,), 16383, jnp.int32))

    def _gather(k, r):
        pltpu.async_copy(t_hbm.at[srcu[k]], rows[r], sg[r])

    def _scatter(r, k):
        pltpu.sync_copy(rows[r], acc_sh.at[dstu[k]], add=True)

    pltpu.sync_copy(pkp.at[wid], pk_v)
    plsc.subcore_barrier()
    _unpack(0, 0)
    _gather(0, 0)
    _unpack(1, 1)

    def body(jj, carry):
        j0 = 2 * jj
        _gather(1, 1)
        _wait(rows[0], sg[0])
        _scatter(0, 0)
        _unpack(j0 + 2, 0)
        _gather(0, 0)
        _wait(rows[1], sg[1])
        _scatter(1, 1)
        _unpack(j0 + 3, 1)
        return carry

    lax.fori_loop(0, CH // 2 - 1, body, 0)
    _gather(1, 1)
    _wait(rows[0], sg[0])
    _scatter(0, 0)
    _wait(rows[1], sg[1])
    _scatter(1, 1)
    plsc.subcore_barrier()
    pltpu.sync_copy(acc_sh.at[pl.ds(s * RPT, RPT)], out.at[c].at[pl.ds(s * RPT, RPT)])

    @pl.when(s == NS - 1)
    def _out_tail():
        pltpu.sync_copy(acc_sh.at[pl.ds(NS * RPT, TAIL)],
                        out.at[c].at[pl.ds(NS * RPT, TAIL)])


def _tc_dis_body(degp_ref, out_ref):
    dp = degp_ref[0]
    d = dp[0:1, :] + dp[1:2, :] - 1.0
    dis = lax.rsqrt(jnp.maximum(d, 1e-12))
    bc = jnp.broadcast_to(dis, (128, 128))
    i0 = lax.broadcasted_iota(jnp.int32, (128, 128), 0)
    i1 = lax.broadcasted_iota(jnp.int32, (128, 128), 1)
    diag = jnp.where(i0 == i1, bc, 0.0)
    out_ref[...] = jnp.dot(diag, jnp.ones((128, 128), jnp.float32),
                           preferred_element_type=jnp.float32, precision=jax.lax.Precision.HIGHEST)


def _tc_dis(degp):
    return pl.pallas_call(
        _tc_dis_body,
        grid=(NPD // 128,),
        in_specs=[pl.BlockSpec((1, 2, 128), lambda r: (r, 0, 0))],
        out_specs=pl.BlockSpec((128, 128), lambda r: (r, 0)),
        out_shape=jax.ShapeDtypeStruct((NPD, D), jnp.float32),
    )(degp)


def _tc_in_body(x_ref, w_ref, dis_ref, out_ref):
    out_ref[...] = dis_ref[...] * jnp.dot(
        x_ref[...], w_ref[...], preferred_element_type=jnp.float32, precision=jax.lax.Precision.HIGHEST)


def _tc_in(x, w, dis):
    return pl.pallas_call(
        _tc_in_body,
        out_shape=jax.ShapeDtypeStruct((N, D), jnp.float32),
    )(x, w, dis)


def _bn_relu(a_ref, tp_ref, dis_ref, b_ref, g_ref, be_ref):
    dis = dis_ref[...]
    u = dis * (a_ref[0] + a_ref[1] - tp_ref[...]) + b_ref[...]
    mu = jnp.mean(u, axis=0, keepdims=True)
    var = jnp.mean((u - mu) ** 2, axis=0, keepdims=True)
    y = (u - mu) * lax.rsqrt(var + 1e-5) * g_ref[...] + be_ref[...]
    return jnp.maximum(y, 0.0), dis


def _tc_mid_body(a_ref, tp_ref, dis_ref, b_ref, g_ref, be_ref, w_ref, out_ref):
    y, dis = _bn_relu(a_ref, tp_ref, dis_ref, b_ref, g_ref, be_ref)
    out_ref[...] = dis * jnp.dot(y, w_ref[...], preferred_element_type=jnp.float32, precision=jax.lax.Precision.HIGHEST)


def _tc_mid(a, tp, dis, b, g, be, w):
    return pl.pallas_call(
        _tc_mid_body,
        out_shape=jax.ShapeDtypeStruct((N, D), jnp.float32),
    )(a, tp, dis, b.reshape(1, D), g.reshape(1, D), be.reshape(1, D), w)


def _tc_fin_body(a_ref, tp_ref, dis_ref, b_ref, g_ref, be_ref,
                 mw0_ref, mb0_ref, mw1_ref, mb1_ref, out_ref):
    y, _ = _bn_relu(a_ref, tp_ref, dis_ref, b_ref, g_ref, be_ref)
    z = jnp.maximum(
        jnp.dot(y, mw0_ref[...], preferred_element_type=jnp.float32, precision=jax.lax.Precision.HIGHEST) + mb0_ref[...],
        0.0)
    out_ref[...] = jnp.dot(z, mw1_ref[...],
                           preferred_element_type=jnp.float32, precision=jax.lax.Precision.HIGHEST) + mb1_ref[...]


def _tc_fin(a, tp, dis, b, g, be, mw0, mb0, mw1, mb1):
    return pl.pallas_call(
        _tc_fin_body,
        out_shape=jax.ShapeDtypeStruct((N, D), jnp.float32),
    )(a, tp, dis, b.reshape(1, D), g.reshape(1, D), be.reshape(1, D),
      mw0, mb0.reshape(1, D), mw1, mb1.reshape(1, D))


def kernel(x, edge_index, W0, b0, g0, be0, W1, b1, g1, be1, W2, b2, g2, be2,
           mW0, mb0, mW1, mb1):
    src, dst = edge_index[0], edge_index[1]
    pad = CAP - E
    srcf = jnp.concatenate([src, jnp.zeros((pad,), src.dtype)])
    dstf = jnp.concatenate([dst, jnp.full((pad,), N, dst.dtype)])
    dstp = dstf.reshape(NW, CH, B)
    pkp = (srcf * 16384 + dstf).reshape(NW, CH, B)

    degp = _sc_degree(dstp)
    dis = _tc_dis(jnp.transpose(degp, (1, 0, 2)))[:N]

    t = _tc_in(x, W0, dis)
    a = _sc_aggregate(t, pkp)
    t1 = _tc_mid(a, t, dis, b0, g0, be0, W1)
    a1 = _sc_aggregate(t1, pkp)
    t2 = _tc_mid(a1, t1, dis, b1, g1, be1, W2)
    a2 = _sc_aggregate(t2, pkp)
    return _tc_fin(a2, t2, dis, b2, g2, be2, mW0, mb0, mW1, mb1)


# speedup vs baseline: 1.1977x; 1.1977x over previous
import functools

import jax
import jax.numpy as jnp
from jax import lax
from jax.experimental import pallas as pl
from jax.experimental.pallas import tpu as pltpu
from jax.experimental.pallas import tpu_sc as plsc

N = 10000
D = 128
E = 320000

NC = 2
NS = 16
NW = NC * NS
B = 128
CH = -(-E // (NW * B))
CAP = NW * CH * B
CH0 = 57
CH1 = 2 * CH - CH0
CHM = max(CH0, CH1)
NPR = N + 8
NPD = NS * 8 * 128
DPT = NPD // NS
RPT = (N // NS) // 8 * 8
TAIL = N - NS * RPT

_mesh = plsc.VectorSubcoreMesh(core_axis_name="c", subcore_axis_name="s")


@functools.partial(
    pl.kernel,
    out_type=jax.ShapeDtypeStruct((NC, NPD // 128, 128), jnp.float32),
    mesh=_mesh,
    scratch_types=[
        pltpu.VMEM((CH, B), jnp.int32),
        pltpu.VMEM((DPT,), jnp.float32),
        pltpu.VMEM((B,), jnp.float32),
        pltpu.VMEM((DPT // 128, 128), jnp.float32),
        pltpu.VMEM_SHARED((NPD,), jnp.float32),
    ],
)
def _sc_degree(dstp, out, dst_v, ones_i, ones_b, stage_v, deg_sh):
    c = lax.axis_index("c")
    s = lax.axis_index("s")
    wid = s * NC + c
    for i in range(DPT // 16):
        ones_i[pl.ds(i * 16, 16)] = jnp.full((16,), 1.0, jnp.float32)
    for i in range(B // 16):
        ones_b[pl.ds(i * 16, 16)] = jnp.full((16,), 1.0, jnp.float32)
    pltpu.sync_copy(dstp.at[wid], dst_v)
    pltpu.sync_copy(ones_i, deg_sh.at[pl.ds(s * DPT, DPT)])
    plsc.subcore_barrier()

    def body(j, carry):
        pltpu.sync_copy(ones_b, deg_sh.at[dst_v.at[j]], add=True)
        return carry

    lax.fori_loop(0, CH, body, 0)
    plsc.subcore_barrier()
    for t in range(DPT // 128):
        pltpu.sync_copy(deg_sh.at[pl.ds(s * DPT + t * 128, 128)], stage_v.at[t])
    pltpu.sync_copy(stage_v, out.at[c].at[pl.ds(s * (DPT // 128), DPT // 128)])


@functools.partial(
    pl.kernel,
    out_type=jax.ShapeDtypeStruct((NC, N, D), jnp.float32),
    mesh=_mesh,
    scratch_types=[
        pltpu.VMEM((CHM, B), jnp.int32),
        pltpu.VMEM((CHM, B), jnp.int32),
        pltpu.VMEM((B, D), jnp.float32),
        pltpu.VMEM_SHARED((NPR, D), jnp.float32),
        pltpu.SemaphoreType.DMA,
    ],
)
def _sc_aggregate(t_hbm, srcp0, dstp0, srcp1, dstp1, out, src_v, dst_v, rows,
                  acc_sh, sg):
    c = lax.axis_index("c")
    s = lax.axis_index("s")
    pltpu.sync_copy(t_hbm.at[pl.ds(s * RPT, RPT)], acc_sh.at[pl.ds(s * RPT, RPT)])

    @pl.when(s == NS - 1)
    def _init_tail():
        pltpu.sync_copy(t_hbm.at[pl.ds(NS * RPT, TAIL)],
                        acc_sh.at[pl.ds(NS * RPT, TAIL)])

    @pl.when(c == 0)
    def _load_idx0():
        pltpu.sync_copy(srcp0.at[s], src_v.at[pl.ds(0, CH0)])
        pltpu.sync_copy(dstp0.at[s], dst_v.at[pl.ds(0, CH0)])

    @pl.when(c == 1)
    def _load_idx1():
        pltpu.sync_copy(srcp1.at[s], src_v.at[pl.ds(0, CH1)])
        pltpu.sync_copy(dstp1.at[s], dst_v.at[pl.ds(0, CH1)])

    plsc.subcore_barrier()

    def body(j, carry):
        pltpu.async_copy(t_hbm.at[src_v.at[j]], rows, sg).wait()
        pltpu.sync_copy(rows, acc_sh.at[dst_v.at[j]], add=True)
        return carry

    lax.fori_loop(0, jnp.where(c == 0, CH0, CH1), body, 0)
    plsc.subcore_barrier()
    pltpu.sync_copy(acc_sh.at[pl.ds(s * RPT, RPT)], out.at[c].at[pl.ds(s * RPT, RPT)])

    @pl.when(s == NS - 1)
    def _out_tail():
        pltpu.sync_copy(acc_sh.at[pl.ds(NS * RPT, TAIL)],
                        out.at[c].at[pl.ds(NS * RPT, TAIL)])


def _tc_dis_body(degp_ref, out_ref):
    dp = degp_ref[0]
    d = dp[0:1, :] + dp[1:2, :] - 1.0
    dis = lax.rsqrt(jnp.maximum(d, 1e-12))
    bc = jnp.broadcast_to(dis, (128, 128))
    i0 = lax.broadcasted_iota(jnp.int32, (128, 128), 0)
    i1 = lax.broadcasted_iota(jnp.int32, (128, 128), 1)
    diag = jnp.where(i0 == i1, bc, 0.0)
    out_ref[...] = jnp.dot(diag, jnp.ones((128, 128), jnp.float32),
                           preferred_element_type=jnp.float32, precision=jax.lax.Precision.HIGHEST)


def _tc_dis(degp):
    return pl.pallas_call(
        _tc_dis_body,
        grid=(NPD // 128,),
        in_specs=[pl.BlockSpec((1, 2, 128), lambda r: (r, 0, 0))],
        out_specs=pl.BlockSpec((128, 128), lambda r: (r, 0)),
        out_shape=jax.ShapeDtypeStruct((NPD, D), jnp.float32),
    )(degp)


def _tc_in_body(x_ref, w_ref, dis_ref, out_ref):
    out_ref[...] = dis_ref[...] * jnp.dot(
        x_ref[...], w_ref[...], preferred_element_type=jnp.float32, precision=jax.lax.Precision.HIGHEST)


def _tc_in(x, w, dis):
    return pl.pallas_call(
        _tc_in_body,
        out_shape=jax.ShapeDtypeStruct((N, D), jnp.float32),
    )(x, w, dis)


def _bn_relu(a_ref, tp_ref, dis_ref, b_ref, g_ref, be_ref):
    dis = dis_ref[...]
    u = dis * (a_ref[0] + a_ref[1] - tp_ref[...]) + b_ref[...]
    mu = jnp.mean(u, axis=0, keepdims=True)
    var = jnp.mean((u - mu) ** 2, axis=0, keepdims=True)
    y = (u - mu) * lax.rsqrt(var + 1e-5) * g_ref[...] + be_ref[...]
    return jnp.maximum(y, 0.0), dis


def _tc_mid_body(a_ref, tp_ref, dis_ref, b_ref, g_ref, be_ref, w_ref, out_ref):
    y, dis = _bn_relu(a_ref, tp_ref, dis_ref, b_ref, g_ref, be_ref)
    out_ref[...] = dis * jnp.dot(y, w_ref[...], preferred_element_type=jnp.float32, precision=jax.lax.Precision.HIGHEST)


def _tc_mid(a, tp, dis, b, g, be, w):
    return pl.pallas_call(
        _tc_mid_body,
        out_shape=jax.ShapeDtypeStruct((N, D), jnp.float32),
    )(a, tp, dis, b.reshape(1, D), g.reshape(1, D), be.reshape(1, D), w)


def _tc_fin_body(a_ref, tp_ref, dis_ref, b_ref, g_ref, be_ref,
                 mw0_ref, mb0_ref, mw1_ref, mb1_ref, out_ref):
    y, _ = _bn_relu(a_ref, tp_ref, dis_ref, b_ref, g_ref, be_ref)
    z = jnp.maximum(
        jnp.dot(y, mw0_ref[...], preferred_element_type=jnp.float32, precision=jax.lax.Precision.HIGHEST) + mb0_ref[...],
        0.0)
    out_ref[...] = jnp.dot(z, mw1_ref[...],
                           preferred_element_type=jnp.float32, precision=jax.lax.Precision.HIGHEST) + mb1_ref[...]


def _tc_fin(a, tp, dis, b, g, be, mw0, mb0, mw1, mb1):
    return pl.pallas_call(
        _tc_fin_body,
        out_shape=jax.ShapeDtypeStruct((N, D), jnp.float32),
    )(a, tp, dis, b.reshape(1, D), g.reshape(1, D), be.reshape(1, D),
      mw0, mb0.reshape(1, D), mw1, mb1.reshape(1, D))


def kernel(x, edge_index, W0, b0, g0, be0, W1, b1, g1, be1, W2, b2, g2, be2,
           mW0, mb0, mW1, mb1):
    src, dst = edge_index[0], edge_index[1]
    pad = CAP - E
    srcf = jnp.concatenate([src, jnp.zeros((pad,), src.dtype)])
    dstf = jnp.concatenate([dst, jnp.full((pad,), N, dst.dtype)])
    dstp = dstf.reshape(NW, CH, B)
    e0 = NS * CH0 * B
    srcp0 = srcf[:e0].reshape(NS, CH0, B)
    dstp0 = dstf[:e0].reshape(NS, CH0, B)
    srcp1 = srcf[e0:].reshape(NS, CH1, B)
    dstp1 = dstf[e0:].reshape(NS, CH1, B)

    degp = _sc_degree(dstp)
    dis = _tc_dis(jnp.transpose(degp, (1, 0, 2)))[:N]

    t = _tc_in(x, W0, dis)
    a = _sc_aggregate(t, srcp0, dstp0, srcp1, dstp1)
    t1 = _tc_mid(a, t, dis, b0, g0, be0, W1)
    a1 = _sc_aggregate(t1, srcp0, dstp0, srcp1, dstp1)
    t2 = _tc_mid(a1, t1, dis, b1, g1, be1, W2)
    a2 = _sc_aggregate(t2, srcp0, dstp0, srcp1, dstp1)
    return _tc_fin(a2, t2, dis, b2, g2, be2, mW0, mb0, mW1, mb1)


# speedup vs baseline: 1.4353x; 1.1984x over previous
import functools

import jax
import jax.numpy as jnp
from jax import lax
from jax.experimental import pallas as pl
from jax.experimental.pallas import tpu as pltpu
from jax.experimental.pallas import tpu_sc as plsc

N = 10000
D = 128
E = 320000

NC = 2
NS = 16
NW = NC * NS
B = 128
CH = -(-E // (NW * B))
CAP = NW * CH * B
CH0 = 101
CH1 = 2 * CH - CH0
CHM = max(CH0, CH1)
NPR = N + 8
NPD = NS * 8 * 128
DPT = NPD // NS
RPT = (N // NS) // 8 * 8
TAIL = N - NS * RPT

_mesh = plsc.VectorSubcoreMesh(core_axis_name="c", subcore_axis_name="s")


@functools.partial(
    pl.kernel,
    out_type=jax.ShapeDtypeStruct((NC, NPD // 128, 128), jnp.float32),
    mesh=_mesh,
    scratch_types=[
        pltpu.VMEM((CH, B), jnp.int32),
        pltpu.VMEM((DPT,), jnp.float32),
        pltpu.VMEM((B,), jnp.float32),
        pltpu.VMEM((DPT // 128, 128), jnp.float32),
        pltpu.VMEM_SHARED((NPD,), jnp.float32),
    ],
)
def _sc_degree(dstp, out, dst_v, ones_i, ones_b, stage_v, deg_sh):
    c = lax.axis_index("c")
    s = lax.axis_index("s")
    wid = s * NC + c
    for i in range(DPT // 16):
        ones_i[pl.ds(i * 16, 16)] = jnp.full((16,), 1.0, jnp.float32)
    for i in range(B // 16):
        ones_b[pl.ds(i * 16, 16)] = jnp.full((16,), 1.0, jnp.float32)
    pltpu.sync_copy(dstp.at[wid], dst_v)
    pltpu.sync_copy(ones_i, deg_sh.at[pl.ds(s * DPT, DPT)])
    plsc.subcore_barrier()

    def body(j, carry):
        pltpu.sync_copy(ones_b, deg_sh.at[dst_v.at[j]], add=True)
        return carry

    lax.fori_loop(0, CH, body, 0)
    plsc.subcore_barrier()
    for t in range(DPT // 128):
        pltpu.sync_copy(deg_sh.at[pl.ds(s * DPT + t * 128, 128)], stage_v.at[t])
    pltpu.sync_copy(stage_v, out.at[c].at[pl.ds(s * (DPT // 128), DPT // 128)])


@functools.partial(
    pl.kernel,
    out_type=jax.ShapeDtypeStruct((NC, N, D), jnp.float32),
    mesh=_mesh,
    scratch_types=[
        pltpu.VMEM((CHM, B), jnp.int32),
        pltpu.VMEM((CHM, B), jnp.int32),
        pltpu.VMEM((B, D), jnp.float32),
        pltpu.VMEM_SHARED((NPR, D), jnp.float32),
        pltpu.SemaphoreType.DMA,
    ],
)
def _sc_aggregate(t_hbm, srcp0, dstp0, srcp1, dstp1, out, src_v, dst_v, rows,
                  acc_sh, sg):
    c = lax.axis_index("c")
    s = lax.axis_index("s")
    pltpu.sync_copy(t_hbm.at[pl.ds(s * RPT, RPT)], acc_sh.at[pl.ds(s * RPT, RPT)])

    @pl.when(s == NS - 1)
    def _init_tail():
        pltpu.sync_copy(t_hbm.at[pl.ds(NS * RPT, TAIL)],
                        acc_sh.at[pl.ds(NS * RPT, TAIL)])

    @pl.when(c == 0)
    def _load_idx0():
        pltpu.sync_copy(srcp0.at[s], src_v.at[pl.ds(0, CH0)])
        pltpu.sync_copy(dstp0.at[s], dst_v.at[pl.ds(0, CH0)])

    @pl.when(c == 1)
    def _load_idx1():
        pltpu.sync_copy(srcp1.at[s], src_v.at[pl.ds(0, CH1)])
        pltpu.sync_copy(dstp1.at[s], dst_v.at[pl.ds(0, CH1)])

    plsc.subcore_barrier()

    def body(j, carry):
        pltpu.async_copy(t_hbm.at[src_v.at[j]], rows, sg).wait()
        pltpu.sync_copy(rows, acc_sh.at[dst_v.at[j]], add=True)
        return carry

    lax.fori_loop(0, jnp.where(c == 0, CH0, CH1), body, 0)
    plsc.subcore_barrier()
    pltpu.sync_copy(acc_sh.at[pl.ds(s * RPT, RPT)], out.at[c].at[pl.ds(s * RPT, RPT)])

    @pl.when(s == NS - 1)
    def _out_tail():
        pltpu.sync_copy(acc_sh.at[pl.ds(NS * RPT, TAIL)],
                        out.at[c].at[pl.ds(NS * RPT, TAIL)])


def _tc_dis_body(degp_ref, out_ref):
    dp = degp_ref[0]
    d = dp[0:1, :] + dp[1:2, :] - 1.0
    dis = lax.rsqrt(jnp.maximum(d, 1e-12))
    bc = jnp.broadcast_to(dis, (128, 128))
    i0 = lax.broadcasted_iota(jnp.int32, (128, 128), 0)
    i1 = lax.broadcasted_iota(jnp.int32, (128, 128), 1)
    diag = jnp.where(i0 == i1, bc, 0.0)
    out_ref[...] = jnp.dot(diag, jnp.ones((128, 128), jnp.float32),
                           preferred_element_type=jnp.float32, precision=jax.lax.Precision.HIGHEST)


def _tc_dis(degp):
    return pl.pallas_call(
        _tc_dis_body,
        grid=(NPD // 128,),
        in_specs=[pl.BlockSpec((1, 2, 128), lambda r: (r, 0, 0))],
        out_specs=pl.BlockSpec((128, 128), lambda r: (r, 0)),
        out_shape=jax.ShapeDtypeStruct((NPD, D), jnp.float32),
    )(degp)


def _tc_in_body(x_ref, w_ref, dis_ref, out_ref):
    out_ref[...] = dis_ref[...] * jnp.dot(
        x_ref[...], w_ref[...], preferred_element_type=jnp.float32, precision=jax.lax.Precision.HIGHEST)


def _tc_in(x, w, dis):
    return pl.pallas_call(
        _tc_in_body,
        out_shape=jax.ShapeDtypeStruct((N, D), jnp.float32),
    )(x, w, dis)


def _bn_relu(a_ref, tp_ref, dis_ref, b_ref, g_ref, be_ref):
    dis = dis_ref[...]
    u = dis * (a_ref[0] + a_ref[1] - tp_ref[...]) + b_ref[...]
    mu = jnp.mean(u, axis=0, keepdims=True)
    var = jnp.mean((u - mu) ** 2, axis=0, keepdims=True)
    y = (u - mu) * lax.rsqrt(var + 1e-5) * g_ref[...] + be_ref[...]
    return jnp.maximum(y, 0.0), dis


def _tc_mid_body(a_ref, tp_ref, dis_ref, b_ref, g_ref, be_ref, w_ref, out_ref):
    y, dis = _bn_relu(a_ref, tp_ref, dis_ref, b_ref, g_ref, be_ref)
    out_ref[...] = dis * jnp.dot(y, w_ref[...], preferred_element_type=jnp.float32, precision=jax.lax.Precision.HIGHEST)


def _tc_mid(a, tp, dis, b, g, be, w):
    return pl.pallas_call(
        _tc_mid_body,
        out_shape=jax.ShapeDtypeStruct((N, D), jnp.float32),
    )(a, tp, dis, b.reshape(1, D), g.reshape(1, D), be.reshape(1, D), w)


def _tc_fin_body(a_ref, tp_ref, dis_ref, b_ref, g_ref, be_ref,
                 mw0_ref, mb0_ref, mw1_ref, mb1_ref, out_ref):
    y, _ = _bn_relu(a_ref, tp_ref, dis_ref, b_ref, g_ref, be_ref)
    z = jnp.maximum(
        jnp.dot(y, mw0_ref[...], preferred_element_type=jnp.float32, precision=jax.lax.Precision.HIGHEST) + mb0_ref[...],
        0.0)
    out_ref[...] = jnp.dot(z, mw1_ref[...],
                           preferred_element_type=jnp.float32, precision=jax.lax.Precision.HIGHEST) + mb1_ref[...]


def _tc_fin(a, tp, dis, b, g, be, mw0, mb0, mw1, mb1):
    return pl.pallas_call(
        _tc_fin_body,
        out_shape=jax.ShapeDtypeStruct((N, D), jnp.float32),
    )(a, tp, dis, b.reshape(1, D), g.reshape(1, D), be.reshape(1, D),
      mw0, mb0.reshape(1, D), mw1, mb1.reshape(1, D))


def kernel(x, edge_index, W0, b0, g0, be0, W1, b1, g1, be1, W2, b2, g2, be2,
           mW0, mb0, mW1, mb1):
    src, dst = edge_index[0], edge_index[1]
    pad = CAP - E
    srcf = jnp.concatenate([src, jnp.zeros((pad,), src.dtype)])
    dstf = jnp.concatenate([dst, jnp.full((pad,), N, dst.dtype)])
    dstp = dstf.reshape(NW, CH, B)
    e0 = NS * CH0 * B
    srcp0 = srcf[:e0].reshape(NS, CH0, B)
    dstp0 = dstf[:e0].reshape(NS, CH0, B)
    srcp1 = srcf[e0:].reshape(NS, CH1, B)
    dstp1 = dstf[e0:].reshape(NS, CH1, B)

    degp = _sc_degree(dstp)
    dis = _tc_dis(jnp.transpose(degp, (1, 0, 2)))[:N]

    t = _tc_in(x, W0, dis)
    a = _sc_aggregate(t, srcp0, dstp0, srcp1, dstp1)
    t1 = _tc_mid(a, t, dis, b0, g0, be0, W1)
    a1 = _sc_aggregate(t1, srcp0, dstp0, srcp1, dstp1)
    t2 = _tc_mid(a1, t1, dis, b1, g1, be1, W2)
    a2 = _sc_aggregate(t2, srcp0, dstp0, srcp1, dstp1)
    return _tc_fin(a2, t2, dis, b2, g2, be2, mW0, mb0, mW1, mb1)
